# baseline (speedup 1.0000x reference)
import functools

import jax
import jax.numpy as jnp
from jax import lax
from jax.experimental import pallas as pl
from jax.experimental.pallas import tpu as pltpu
from jax.experimental.pallas import tpu_sc as plsc

_D = 128
_B = 16384
_BLK = 4096
_NC = 2
_NS = 16
_NW = _NC * _NS
_CHUNK = _B // _NW
_NQ = 4
_KEYS_PER_SUB = 25600
_TAB = 4 * _KEYS_PER_SUB
_QVEC = _B // _NQ // 16
_QLEN = _B // _NQ


def _sc_body(hidden, x, idx, h_old_out, xp_out,
             idx_v, idx_chunk_v, table_v, p0_v, p1_v, p2_v, p3_v,
             rows_v, xrows_v,
             tab0, tab1, tab2, tab3,
             s1, s2, s3, s4, s5):
    c = lax.axis_index("c")
    s = lax.axis_index("s")
    wid = s * _NC + c
    base = wid * _CHUNK
    half = _CHUNK // 2
    q = s & 3
    r = s >> 2

    pltpu.sync_copy(idx.at[pl.ds(base, _CHUNK)], idx_chunk_v)
    g0 = pltpu.async_copy(hidden.at[idx_chunk_v.at[pl.ds(0, half)]], rows_v, s1)
    g1 = pltpu.async_copy(hidden.at[idx_chunk_v.at[pl.ds(half, half)]], xrows_v, s2)

    pltpu.sync_copy(idx.at[pl.ds(q * _QLEN, _QLEN)], idx_v)

    neg1 = jnp.full((16,), -1, jnp.int32)

    def init_step(i, carry):
        table_v[pl.ds(i * 16, 16)] = neg1
        return carry

    with jax.named_scope("tab_init"):
        lax.fori_loop(0, _KEYS_PER_SUB // 16, init_step, 0)

    lo = r * _KEYS_PER_SUB
    jbase = q * _QLEN
    lane = lax.iota(jnp.int32, 16)
    lane_next = jnp.minimum(lane + 1, 15)
    is_lane15 = lane == 15

    def scan_step(i, carry):
        idx16 = idx_v[pl.ds(i * 16, 16)]
        comp = idx16 * _B + jbase + i * 16 + lane
        comp_s, _ = plsc.sort_key_val(comp, comp)
        key = lax.shift_right_arithmetic(comp_s, 14)
        nkey = key.at[lane_next].get(mode="promise_in_bounds")
        jj = comp_s & (_B - 1)
        last = (key != nkey) | is_lane15
        inr = (key >= lo) & (key < lo + _KEYS_PER_SUB)
        m = last & inr
        loc = jnp.where(m, key - lo, 0)
        plsc.store_scatter(table_v, [loc], jj, mask=m)
        return carry

    with jax.named_scope("dedup_scan"):
        lax.fori_loop(0, _QVEC, scan_step, 0, unroll=4)

    for qi, tq in enumerate((tab0, tab1, tab2, tab3)):
        @pl.when(q == qi)
        def _publish(tq=tq):
            pltpu.sync_copy(table_v, tq.at[pl.ds(lo, _KEYS_PER_SUB)])

    with jax.named_scope("h_drain"):
        g0.wait()
        w0 = pltpu.async_copy(rows_v, h_old_out.at[pl.ds(base, half)], s3)
        g1.wait()
        w1 = pltpu.async_copy(xrows_v, h_old_out.at[pl.ds(base + half, half)], s4)

    with jax.named_scope("pos_merge"):
        plsc.subcore_barrier()

        descs = []
        for tq, pq in ((tab0, p0_v), (tab1, p1_v), (tab2, p2_v), (tab3, p3_v)):
            descs.append(pltpu.async_copy(tq.at[idx_chunk_v], pq, s5))
        for d in descs:
            d.wait()

        def merge_step(i, carry):
            sl = pl.ds(i * 16, 16)
            m01 = jnp.maximum(p0_v[sl], p1_v[sl])
            m23 = jnp.maximum(p2_v[sl], p3_v[sl])
            p0_v[sl] = jnp.maximum(m01, m23)
            return carry

        lax.fori_loop(0, _CHUNK // 16, merge_step, 0)

    with jax.named_scope("xp_phase"):
        w0.wait()
        xa = pltpu.async_copy(x.at[p0_v.at[pl.ds(0, half)]], rows_v, s1)
        w1.wait()
        xb = pltpu.async_copy(x.at[p0_v.at[pl.ds(half, half)]], xrows_v, s2)
        xa.wait()
        wxa = pltpu.async_copy(rows_v, xp_out.at[pl.ds(base, half)], s3)
        xb.wait()
        pltpu.sync_copy(xrows_v, xp_out.at[pl.ds(base + half, half)])
        wxa.wait()


def _sc_gather(hidden, x, idx):
    mesh = plsc.VectorSubcoreMesh(core_axis_name="c", subcore_axis_name="s")
    f = functools.partial(
        pl.kernel,
        out_type=[
            jax.ShapeDtypeStruct((_B, _D), jnp.float32),
            jax.ShapeDtypeStruct((_B, _D), jnp.float32),
        ],
        mesh=mesh,
        scratch_types=[
            pltpu.VMEM((_QLEN,), jnp.int32),
            pltpu.VMEM((_CHUNK,), jnp.int32),
            pltpu.VMEM((_KEYS_PER_SUB,), jnp.int32),
            pltpu.VMEM((_CHUNK,), jnp.int32),
            pltpu.VMEM((_CHUNK,), jnp.int32),
            pltpu.VMEM((_CHUNK,), jnp.int32),
            pltpu.VMEM((_CHUNK,), jnp.int32),
            pltpu.VMEM((_CHUNK // 2, _D), jnp.float32),
            pltpu.VMEM((_CHUNK // 2, _D), jnp.float32),
            pltpu.VMEM_SHARED((_TAB,), jnp.int32),
            pltpu.VMEM_SHARED((_TAB,), jnp.int32),
            pltpu.VMEM_SHARED((_TAB,), jnp.int32),
            pltpu.VMEM_SHARED((_TAB,), jnp.int32),
            pltpu.SemaphoreType.DMA,
            pltpu.SemaphoreType.DMA,
            pltpu.SemaphoreType.DMA,
            pltpu.SemaphoreType.DMA,
            pltpu.SemaphoreType.DMA,
        ],
        compiler_params=pltpu.CompilerParams(needs_layout_passes=False),
    )(_sc_body)
    return f(hidden, x, idx)


def _gru_body(xp_ref, h_ref, w_all_ref, b_all_ref, out_ref):
    xp = xp_ref[...]
    h = h_ref[...]
    xh = jnp.concatenate([xp, h], axis=1).astype(jnp.bfloat16)
    g = jnp.dot(xh, w_all_ref[...], preferred_element_type=jnp.float32) + b_all_ref[...]
    r = jax.nn.sigmoid(g[:, :_D])
    z = jax.nn.sigmoid(g[:, _D:2 * _D])
    n = jnp.tanh(g[:, 2 * _D:3 * _D] + r * g[:, 3 * _D:])
    out_ref[...] = (1.0 - z) * n + z * h


def _gru_pallas(xp, h_old, w_all, b_all):
    b = xp.shape[0]
    grid = (b // _BLK,)
    return pl.pallas_call(
        _gru_body,
        grid=grid,
        in_specs=[
            pl.BlockSpec((_BLK, _D), lambda i: (i, 0)),
            pl.BlockSpec((_BLK, _D), lambda i: (i, 0)),
            pl.BlockSpec((2 * _D, 4 * _D), lambda i: (0, 0)),
            pl.BlockSpec((1, 4 * _D), lambda i: (0, 0)),
        ],
        out_specs=pl.BlockSpec((_BLK, _D), lambda i: (i, 0)),
        out_shape=jax.ShapeDtypeStruct((b, _D), jnp.float32),
    )(xp, h_old, w_all, b_all)


def kernel(hidden, x, idx, W_ih, W_hh, b_ih, b_hh):
    idx = idx.astype(jnp.int32)
    h_old, xp = _sc_gather(hidden, x, idx)
    zero = jnp.zeros((_D, _D), jnp.float32)
    w_all = jnp.concatenate([
        jnp.concatenate([W_ih[:2 * _D].T, W_ih[2 * _D:].T, zero], axis=1),
        jnp.concatenate([W_hh[:2 * _D].T, zero, W_hh[2 * _D:].T], axis=1),
    ], axis=0).astype(jnp.bfloat16)
    b_all = jnp.concatenate(
        [b_ih[:2 * _D] + b_hh[:2 * _D], b_ih[2 * _D:], b_hh[2 * _D:]])[None, :]
    return _gru_pallas(xp, h_old, w_all, b_all)


# speedup vs baseline: 1.1194x; 1.1194x over previous
import functools

import jax
import jax.numpy as jnp
from jax import lax
from jax.experimental import pallas as pl
from jax.experimental.pallas import tpu as pltpu
from jax.experimental.pallas import tpu_sc as plsc

_D = 128
_B = 16384
_BLK = 4096
_NC = 2
_NS = 16
_NW = _NC * _NS
_CHUNK = _B // _NW
_NQ = 4
_KEYS_PER_SUB = 25600
_TAB = 4 * _KEYS_PER_SUB
_QVEC = _B // _NQ // 16
_QLEN = _B // _NQ


def _sc_body(hidden, x, idx, h_old_out, xp_out,
             idx_v, idx_chunk_v, table_v, p0_v, p1_v, p2_v, p3_v,
             rows_v, xrows_v,
             tab0, tab1, tab2, tab3,
             s1, s2, s3, s4, s5):
    c = lax.axis_index("c")
    s = lax.axis_index("s")
    wid = s * _NC + c
    base = wid * _CHUNK
    half = _CHUNK // 2
    q = s & 3
    r = s >> 2

    pltpu.sync_copy(idx.at[pl.ds(base, _CHUNK)], idx_chunk_v)
    g0 = pltpu.async_copy(hidden.at[idx_chunk_v.at[pl.ds(0, half)]], rows_v, s1)
    g1 = pltpu.async_copy(hidden.at[idx_chunk_v.at[pl.ds(half, half)]], xrows_v, s2)

    pltpu.sync_copy(idx.at[pl.ds(q * _QLEN, _QLEN)], idx_v)

    neg1 = jnp.full((16,), -1, jnp.int32)

    def init_step(i, carry):
        table_v[pl.ds(i * 16, 16)] = neg1
        return carry

    with jax.named_scope("tab_init"):
        lax.fori_loop(0, _KEYS_PER_SUB // 16, init_step, 0, unroll=16)

    lo = r * _KEYS_PER_SUB
    jbase = q * _QLEN
    lane = lax.iota(jnp.int32, 16)
    lane_next = jnp.minimum(lane + 1, 15)
    is_lane15 = lane == 15

    def scan_step(i, carry):
        idx16 = idx_v[pl.ds(i * 16, 16)]
        comp = idx16 * _B + jbase + i * 16 + lane
        comp_s, _ = plsc.sort_key_val(comp, comp)
        key = lax.shift_right_arithmetic(comp_s, 14)
        nkey = key.at[lane_next].get(mode="promise_in_bounds")
        jj = comp_s & (_B - 1)
        last = (key != nkey) | is_lane15
        inr = (key >= lo) & (key < lo + _KEYS_PER_SUB)
        m = last & inr
        loc = jnp.where(m, key - lo, 0)
        plsc.store_scatter(table_v, [loc], jj, mask=m)
        return carry

    with jax.named_scope("dedup_scan"):
        lax.fori_loop(0, _QVEC, scan_step, 0, unroll=4)

    for qi, tq in enumerate((tab0, tab1, tab2, tab3)):
        @pl.when(q == qi)
        def _publish(tq=tq):
            pltpu.sync_copy(table_v, tq.at[pl.ds(lo, _KEYS_PER_SUB)])

    with jax.named_scope("h_drain"):
        g0.wait()
        w0 = pltpu.async_copy(rows_v, h_old_out.at[pl.ds(base, half)], s3)
        g1.wait()
        w1 = pltpu.async_copy(xrows_v, h_old_out.at[pl.ds(base + half, half)], s4)

    with jax.named_scope("pos_merge"):
        plsc.subcore_barrier()

        descs = []
        for tq, pq in ((tab0, p0_v), (tab1, p1_v), (tab2, p2_v), (tab3, p3_v)):
            descs.append(pltpu.async_copy(tq.at[idx_chunk_v], pq, s5))
        for d in descs:
            d.wait()

        def merge_step(i, carry):
            sl = pl.ds(i * 16, 16)
            m01 = jnp.maximum(p0_v[sl], p1_v[sl])
            m23 = jnp.maximum(p2_v[sl], p3_v[sl])
            p0_v[sl] = jnp.maximum(m01, m23)
            return carry

        lax.fori_loop(0, _CHUNK // 16, merge_step, 0)

    with jax.named_scope("xp_phase"):
        w0.wait()
        xa = pltpu.async_copy(x.at[p0_v.at[pl.ds(0, half)]], rows_v, s1)
        w1.wait()
        xb = pltpu.async_copy(x.at[p0_v.at[pl.ds(half, half)]], xrows_v, s2)
        xa.wait()
        wxa = pltpu.async_copy(rows_v, xp_out.at[pl.ds(base, half)], s3)
        xb.wait()
        pltpu.sync_copy(xrows_v, xp_out.at[pl.ds(base + half, half)])
        wxa.wait()


def _sc_gather(hidden, x, idx):
    mesh = plsc.VectorSubcoreMesh(core_axis_name="c", subcore_axis_name="s")
    f = functools.partial(
        pl.kernel,
        out_type=[
            jax.ShapeDtypeStruct((_B, _D), jnp.float32),
            jax.ShapeDtypeStruct((_B, _D), jnp.float32),
        ],
        mesh=mesh,
        scratch_types=[
            pltpu.VMEM((_QLEN,), jnp.int32),
            pltpu.VMEM((_CHUNK,), jnp.int32),
            pltpu.VMEM((_KEYS_PER_SUB,), jnp.int32),
            pltpu.VMEM((_CHUNK,), jnp.int32),
            pltpu.VMEM((_CHUNK,), jnp.int32),
            pltpu.VMEM((_CHUNK,), jnp.int32),
            pltpu.VMEM((_CHUNK,), jnp.int32),
            pltpu.VMEM((_CHUNK // 2, _D), jnp.float32),
            pltpu.VMEM((_CHUNK // 2, _D), jnp.float32),
            pltpu.VMEM_SHARED((_TAB,), jnp.int32),
            pltpu.VMEM_SHARED((_TAB,), jnp.int32),
            pltpu.VMEM_SHARED((_TAB,), jnp.int32),
            pltpu.VMEM_SHARED((_TAB,), jnp.int32),
            pltpu.SemaphoreType.DMA,
            pltpu.SemaphoreType.DMA,
            pltpu.SemaphoreType.DMA,
            pltpu.SemaphoreType.DMA,
            pltpu.SemaphoreType.DMA,
        ],
        compiler_params=pltpu.CompilerParams(needs_layout_passes=False),
    )(_sc_body)
    return f(hidden, x, idx)


def _gru_body(xp_ref, h_ref, w_all_ref, b_all_ref, out_ref):
    xp = xp_ref[...]
    h = h_ref[...]
    xh = jnp.concatenate([xp, h], axis=1).astype(jnp.bfloat16)
    g = jnp.dot(xh, w_all_ref[...], preferred_element_type=jnp.float32) + b_all_ref[...]
    r = jax.nn.sigmoid(g[:, :_D])
    z = jax.nn.sigmoid(g[:, _D:2 * _D])
    n = jnp.tanh(g[:, 2 * _D:3 * _D] + r * g[:, 3 * _D:])
    out_ref[...] = (1.0 - z) * n + z * h


def _gru_pallas(xp, h_old, w_all, b_all):
    b = xp.shape[0]
    grid = (b // _BLK,)
    return pl.pallas_call(
        _gru_body,
        grid=grid,
        in_specs=[
            pl.BlockSpec((_BLK, _D), lambda i: (i, 0)),
            pl.BlockSpec((_BLK, _D), lambda i: (i, 0)),
            pl.BlockSpec((2 * _D, 4 * _D), lambda i: (0, 0)),
            pl.BlockSpec((1, 4 * _D), lambda i: (0, 0)),
        ],
        out_specs=pl.BlockSpec((_BLK, _D), lambda i: (i, 0)),
        out_shape=jax.ShapeDtypeStruct((b, _D), jnp.float32),
    )(xp, h_old, w_all, b_all)


def kernel(hidden, x, idx, W_ih, W_hh, b_ih, b_hh):
    idx = idx.astype(jnp.int32)
    h_old, xp = _sc_gather(hidden, x, idx)
    zero = jnp.zeros((_D, _D), jnp.float32)
    w_all = jnp.concatenate([
        jnp.concatenate([W_ih[:2 * _D].T, W_ih[2 * _D:].T, zero], axis=1),
        jnp.concatenate([W_hh[:2 * _D].T, zero, W_hh[2 * _D:].T], axis=1),
    ], axis=0).astype(jnp.bfloat16)
    b_all = jnp.concatenate(
        [b_ih[:2 * _D] + b_hh[:2 * _D], b_ih[2 * _D:], b_hh[2 * _D:]])[None, :]
    return _gru_pallas(xp, h_old, w_all, b_all)


# speedup vs baseline: 1.1222x; 1.0025x over previous
import functools

import jax
import jax.numpy as jnp
from jax import lax
from jax.experimental import pallas as pl
from jax.experimental.pallas import tpu as pltpu
from jax.experimental.pallas import tpu_sc as plsc

_D = 128
_B = 16384
_BLK = 8192
_NC = 2
_NS = 16
_NW = _NC * _NS
_CHUNK = _B // _NW
_NQ = 4
_KEYS_PER_SUB = 25600
_TAB = 4 * _KEYS_PER_SUB
_QVEC = _B // _NQ // 16
_QLEN = _B // _NQ


def _sc_body(hidden, x, idx, h_old_out, xp_out,
             idx_v, idx_chunk_v, table_v, p0_v, p1_v, p2_v, p3_v,
             rows_v, xrows_v,
             tab0, tab1, tab2, tab3,
             s1, s2, s3, s4, s5):
    c = lax.axis_index("c")
    s = lax.axis_index("s")
    wid = s * _NC + c
    base = wid * _CHUNK
    half = _CHUNK // 2
    q = s & 3
    r = s >> 2

    pltpu.sync_copy(idx.at[pl.ds(base, _CHUNK)], idx_chunk_v)
    g0 = pltpu.async_copy(hidden.at[idx_chunk_v.at[pl.ds(0, half)]], rows_v, s1)
    g1 = pltpu.async_copy(hidden.at[idx_chunk_v.at[pl.ds(half, half)]], xrows_v, s2)

    pltpu.sync_copy(idx.at[pl.ds(q * _QLEN, _QLEN)], idx_v)

    neg1 = jnp.full((16,), -1, jnp.int32)

    def init_step(i, carry):
        table_v[pl.ds(i * 16, 16)] = neg1
        return carry

    with jax.named_scope("tab_init"):
        lax.fori_loop(0, _KEYS_PER_SUB // 16, init_step, 0, unroll=16)

    lo = r * _KEYS_PER_SUB
    jbase = q * _QLEN
    lane = lax.iota(jnp.int32, 16)
    lane_next = jnp.minimum(lane + 1, 15)
    is_lane15 = lane == 15

    def scan_step(i, carry):
        idx16 = idx_v[pl.ds(i * 16, 16)]
        comp = idx16 * _B + jbase + i * 16 + lane
        comp_s, _ = plsc.sort_key_val(comp, comp)
        key = lax.shift_right_arithmetic(comp_s, 14)
        nkey = key.at[lane_next].get(mode="promise_in_bounds")
        jj = comp_s & (_B - 1)
        last = (key != nkey) | is_lane15
        inr = (key >= lo) & (key < lo + _KEYS_PER_SUB)
        m = last & inr
        loc = jnp.where(m, key - lo, 0)
        plsc.store_scatter(table_v, [loc], jj, mask=m)
        return carry

    with jax.named_scope("dedup_scan"):
        lax.fori_loop(0, _QVEC, scan_step, 0, unroll=4)

    for qi, tq in enumerate((tab0, tab1, tab2, tab3)):
        @pl.when(q == qi)
        def _publish(tq=tq):
            pltpu.sync_copy(table_v, tq.at[pl.ds(lo, _KEYS_PER_SUB)])

    with jax.named_scope("h_drain"):
        g0.wait()
        w0 = pltpu.async_copy(rows_v, h_old_out.at[pl.ds(base, half)], s3)
        g1.wait()
        w1 = pltpu.async_copy(xrows_v, h_old_out.at[pl.ds(base + half, half)], s4)

    with jax.named_scope("pos_merge"):
        plsc.subcore_barrier()

        descs = []
        for tq, pq in ((tab0, p0_v), (tab1, p1_v), (tab2, p2_v), (tab3, p3_v)):
            descs.append(pltpu.async_copy(tq.at[idx_chunk_v], pq, s5))
        for d in descs:
            d.wait()

        def merge_step(i, carry):
            sl = pl.ds(i * 16, 16)
            m01 = jnp.maximum(p0_v[sl], p1_v[sl])
            m23 = jnp.maximum(p2_v[sl], p3_v[sl])
            p0_v[sl] = jnp.maximum(m01, m23)
            return carry

        lax.fori_loop(0, _CHUNK // 16, merge_step, 0)

    with jax.named_scope("xp_phase"):
        w0.wait()
        xa = pltpu.async_copy(x.at[p0_v.at[pl.ds(0, half)]], rows_v, s1)
        w1.wait()
        xb = pltpu.async_copy(x.at[p0_v.at[pl.ds(half, half)]], xrows_v, s2)
        xa.wait()
        wxa = pltpu.async_copy(rows_v, xp_out.at[pl.ds(base, half)], s3)
        xb.wait()
        pltpu.sync_copy(xrows_v, xp_out.at[pl.ds(base + half, half)])
        wxa.wait()


def _sc_gather(hidden, x, idx):
    mesh = plsc.VectorSubcoreMesh(core_axis_name="c", subcore_axis_name="s")
    f = functools.partial(
        pl.kernel,
        out_type=[
            jax.ShapeDtypeStruct((_B, _D), jnp.float32),
            jax.ShapeDtypeStruct((_B, _D), jnp.float32),
        ],
        mesh=mesh,
        scratch_types=[
            pltpu.VMEM((_QLEN,), jnp.int32),
            pltpu.VMEM((_CHUNK,), jnp.int32),
            pltpu.VMEM((_KEYS_PER_SUB,), jnp.int32),
            pltpu.VMEM((_CHUNK,), jnp.int32),
            pltpu.VMEM((_CHUNK,), jnp.int32),
            pltpu.VMEM((_CHUNK,), jnp.int32),
            pltpu.VMEM((_CHUNK,), jnp.int32),
            pltpu.VMEM((_CHUNK // 2, _D), jnp.float32),
            pltpu.VMEM((_CHUNK // 2, _D), jnp.float32),
            pltpu.VMEM_SHARED((_TAB,), jnp.int32),
            pltpu.VMEM_SHARED((_TAB,), jnp.int32),
            pltpu.VMEM_SHARED((_TAB,), jnp.int32),
            pltpu.VMEM_SHARED((_TAB,), jnp.int32),
            pltpu.SemaphoreType.DMA,
            pltpu.SemaphoreType.DMA,
            pltpu.SemaphoreType.DMA,
            pltpu.SemaphoreType.DMA,
            pltpu.SemaphoreType.DMA,
        ],
        compiler_params=pltpu.CompilerParams(needs_layout_passes=False),
    )(_sc_body)
    return f(hidden, x, idx)


def _gru_body(xp_ref, h_ref, w_all_ref, b_all_ref, out_ref):
    xp = xp_ref[...]
    h = h_ref[...]
    xh = jnp.concatenate([xp, h], axis=1).astype(jnp.bfloat16)
    g = jnp.dot(xh, w_all_ref[...], preferred_element_type=jnp.float32) + b_all_ref[...]
    r = jax.nn.sigmoid(g[:, :_D])
    z = jax.nn.sigmoid(g[:, _D:2 * _D])
    n = jnp.tanh(g[:, 2 * _D:3 * _D] + r * g[:, 3 * _D:])
    out_ref[...] = (1.0 - z) * n + z * h


def _gru_pallas(xp, h_old, w_all, b_all):
    b = xp.shape[0]
    grid = (b // _BLK,)
    return pl.pallas_call(
        _gru_body,
        grid=grid,
        in_specs=[
            pl.BlockSpec((_BLK, _D), lambda i: (i, 0)),
            pl.BlockSpec((_BLK, _D), lambda i: (i, 0)),
            pl.BlockSpec((2 * _D, 4 * _D), lambda i: (0, 0)),
            pl.BlockSpec((1, 4 * _D), lambda i: (0, 0)),
        ],
        out_specs=pl.BlockSpec((_BLK, _D), lambda i: (i, 0)),
        out_shape=jax.ShapeDtypeStruct((b, _D), jnp.float32),
    )(xp, h_old, w_all, b_all)


def kernel(hidden, x, idx, W_ih, W_hh, b_ih, b_hh):
    idx = idx.astype(jnp.int32)
    h_old, xp = _sc_gather(hidden, x, idx)
    zero = jnp.zeros((_D, _D), jnp.float32)
    w_all = jnp.concatenate([
        jnp.concatenate([W_ih[:2 * _D].T, W_ih[2 * _D:].T, zero], axis=1),
        jnp.concatenate([W_hh[:2 * _D].T, zero, W_hh[2 * _D:].T], axis=1),
    ], axis=0).astype(jnp.bfloat16)
    b_all = jnp.concatenate(
        [b_ih[:2 * _D] + b_hh[:2 * _D], b_ih[2 * _D:], b_hh[2 * _D:]])[None, :]
    return _gru_pallas(xp, h_old, w_all, b_all)
